# baseline (speedup 1.0000x reference)
import jax
import jax.numpy as jnp
from jax import lax
from jax.experimental import pallas as pl
from jax.experimental.pallas import tpu as pltpu
from jax.experimental.pallas import tpu_sc as plsc

NUM_EMBEDDINGS = 1000000
D = 32
B = 16384
L = 50
NW = 32
BAGS_PER_W = B // NW
CHUNK_BAGS = 32
CHUNK_ROWS = CHUNK_BAGS * L
NCHUNK = BAGS_PER_W // CHUNK_BAGS
LANES = 16
DBLK = 8
UNROLL = 5


def _worker(idx_hbm, w_hbm, emb_hbm, out_hbm,
            idx_v, w_v, rows_v, out_v, isem, wsem, gsem, osem):
    cid = lax.axis_index("c")
    sid = lax.axis_index("s")
    wid = cid * 16 + sid
    lanes = lax.iota(jnp.int32, LANES)
    bag0 = wid * BAGS_PER_W

    def idx_copy(c, b):
        return pltpu.make_async_copy(
            idx_hbm.at[pl.ds((bag0 + c * CHUNK_BAGS) * L, CHUNK_ROWS)],
            idx_v[b], isem[b])

    def w_copy(c, b):
        return pltpu.make_async_copy(
            w_hbm.at[pl.ds((bag0 + c * CHUNK_BAGS) * L, CHUNK_ROWS)],
            w_v[b], wsem[b])

    def gather_copy(b):
        return pltpu.make_async_copy(emb_hbm.at[idx_v[b]], rows_v[b], gsem[b])

    def out_copy(c, b):
        return pltpu.make_async_copy(
            out_v[b],
            out_hbm.at[pl.ds(bag0 + c * CHUNK_BAGS, CHUNK_BAGS), :],
            osem[b])

    def compute(b):
        for g in range(CHUNK_BAGS // LANES):
            row_base = g * (LANES * L) + lanes * L
            bag_vec = g * LANES + lanes
            for d0 in range(0, D, DBLK):
                dsp = [jnp.full((LANES,), d, jnp.int32)
                       for d in range(d0, d0 + DBLK)]

                def body(i, accs):
                    new = list(accs)
                    for u in range(UNROLL):
                        r = row_base + (i * UNROLL + u)
                        wl = plsc.load_gather(w_v[b], [r])
                        vs = [plsc.load_gather(rows_v[b], [r, ds])
                              for ds in dsp]
                        for k in range(DBLK):
                            new[k] = new[k] + wl * vs[k]
                    return tuple(new)

                accs = lax.fori_loop(
                    0, L // UNROLL, body,
                    tuple(jnp.zeros((LANES,), jnp.float32)
                          for _ in range(DBLK)))

                for k in range(DBLK):
                    plsc.store_scatter(out_v[b], [bag_vec, dsp[k]], accs[k])

    for b in (0, 1):
        idx_copy(b, b).start()
        idx_copy(b, b).wait()
        gather_copy(b).start()
        w_copy(b, b).start()

    def pair(cp, _):
        for sub in (0, 1):
            c = cp * 2 + sub
            b = sub
            gather_copy(b).wait()

            @pl.when(c + 2 < NCHUNK)
            def _():
                idx_copy(c + 2, b).start()

            @pl.when(c >= 2)
            def _():
                out_copy(c, b).wait()

            w_copy(c, b).wait()
            compute(b)
            out_copy(c, b).start()

            @pl.when(c + 2 < NCHUNK)
            def _():
                w_copy(c + 2, b).start()
                idx_copy(c + 2, b).wait()
                gather_copy(b).start()
        return ()

    lax.fori_loop(0, NCHUNK // 2, pair, ())
    out_copy(NCHUNK - 2, 0).wait()
    out_copy(NCHUNK - 1, 1).wait()


@jax.jit
def kernel(indices, weights, embeddings):
    idx_flat = indices.reshape(-1)
    w_flat = weights.reshape(-1)

    run = pl.kernel(
        _worker,
        out_type=jax.ShapeDtypeStruct((B, D), jnp.float32),
        mesh=plsc.VectorSubcoreMesh(core_axis_name="c", subcore_axis_name="s"),
        compiler_params=pltpu.CompilerParams(
            needs_layout_passes=False, use_tc_tiling_on_sc=False),
        scratch_types=[
            [pltpu.VMEM((CHUNK_ROWS,), jnp.int32) for _ in range(2)],
            [pltpu.VMEM((CHUNK_ROWS,), jnp.float32) for _ in range(2)],
            [pltpu.VMEM((CHUNK_ROWS, D), jnp.float32) for _ in range(2)],
            [pltpu.VMEM((CHUNK_BAGS, D), jnp.float32) for _ in range(2)],
            [pltpu.SemaphoreType.DMA for _ in range(2)],
            [pltpu.SemaphoreType.DMA for _ in range(2)],
            [pltpu.SemaphoreType.DMA for _ in range(2)],
            [pltpu.SemaphoreType.DMA for _ in range(2)],
        ],
    )
    return run(idx_flat, w_flat, embeddings)


# speedup vs baseline: 1.4938x; 1.4938x over previous
import jax
import jax.numpy as jnp
from jax import lax
from jax.experimental import pallas as pl
from jax.experimental.pallas import tpu as pltpu
from jax.experimental.pallas import tpu_sc as plsc

NUM_EMBEDDINGS = 1000000
D = 32
B = 16384
L = 50
NW = 32
BAGS_PER_W = B // NW
CHUNK_BAGS = 32
CHUNK_ROWS = CHUNK_BAGS * L
NCHUNK = BAGS_PER_W // CHUNK_BAGS
LANES = 16
DBLK = 8
UNROLL = 5


def _worker(idx_hbm, w_hbm, emb_hbm, out_hbm,
            idx_v, w_v, rows_v, out_v, isem, wsem, gsem, osem):
    cid = lax.axis_index("c")
    sid = lax.axis_index("s")
    wid = cid * 16 + sid
    lanes = lax.iota(jnp.int32, LANES)
    bag0 = wid * BAGS_PER_W

    def idx_copy(c, b):
        return pltpu.make_async_copy(
            idx_hbm.at[pl.ds((bag0 + c * CHUNK_BAGS) * L, CHUNK_ROWS)],
            idx_v[b], isem[b])

    def w_copy(c, b):
        return pltpu.make_async_copy(
            w_hbm.at[pl.ds((bag0 + c * CHUNK_BAGS) * L, CHUNK_ROWS)],
            w_v[b], wsem[b])

    def gather_copy(b):
        return pltpu.make_async_copy(emb_hbm.at[idx_v[b]], rows_v[b], gsem[b])

    def out_copy(c, b):
        return pltpu.make_async_copy(
            out_v[b],
            out_hbm.at[pl.ds(bag0 + c * CHUNK_BAGS, CHUNK_BAGS), :],
            osem[b])

    def compute(b):
        for g in range(CHUNK_BAGS // LANES):
            row_base = g * (LANES * L) + lanes * L
            bag_vec = g * LANES + lanes
            for d0 in range(0, D, DBLK):
                dsp = [(lanes + (d0 + k)) & (D - 1) for k in range(DBLK)]

                def body(i, accs):
                    new = list(accs)
                    for u in range(UNROLL):
                        r = row_base + (i * UNROLL + u)
                        wl = plsc.load_gather(w_v[b], [r])
                        vs = [plsc.load_gather(rows_v[b], [r, ds])
                              for ds in dsp]
                        for k in range(DBLK):
                            new[k] = new[k] + wl * vs[k]
                    return tuple(new)

                accs = lax.fori_loop(
                    0, L // UNROLL, body,
                    tuple(jnp.zeros((LANES,), jnp.float32)
                          for _ in range(DBLK)))

                for k in range(DBLK):
                    plsc.store_scatter(out_v[b], [bag_vec, dsp[k]], accs[k])

    for b in (0, 1):
        idx_copy(b, b).start()
        idx_copy(b, b).wait()
        gather_copy(b).start()
        w_copy(b, b).start()

    def pair(cp, _):
        for sub in (0, 1):
            c = cp * 2 + sub
            b = sub
            gather_copy(b).wait()

            @pl.when(c + 2 < NCHUNK)
            def _():
                idx_copy(c + 2, b).start()

            @pl.when(c >= 2)
            def _():
                out_copy(c, b).wait()

            w_copy(c, b).wait()
            compute(b)
            out_copy(c, b).start()

            @pl.when(c + 2 < NCHUNK)
            def _():
                w_copy(c + 2, b).start()
                idx_copy(c + 2, b).wait()
                gather_copy(b).start()
        return ()

    lax.fori_loop(0, NCHUNK // 2, pair, ())
    out_copy(NCHUNK - 2, 0).wait()
    out_copy(NCHUNK - 1, 1).wait()


@jax.jit
def kernel(indices, weights, embeddings):
    idx_flat = indices.reshape(-1)
    w_flat = weights.reshape(-1)

    run = pl.kernel(
        _worker,
        out_type=jax.ShapeDtypeStruct((B, D), jnp.float32),
        mesh=plsc.VectorSubcoreMesh(core_axis_name="c", subcore_axis_name="s"),
        compiler_params=pltpu.CompilerParams(
            needs_layout_passes=False, use_tc_tiling_on_sc=False),
        scratch_types=[
            [pltpu.VMEM((CHUNK_ROWS,), jnp.int32) for _ in range(2)],
            [pltpu.VMEM((CHUNK_ROWS,), jnp.float32) for _ in range(2)],
            [pltpu.VMEM((CHUNK_ROWS, D), jnp.float32) for _ in range(2)],
            [pltpu.VMEM((CHUNK_BAGS, D), jnp.float32) for _ in range(2)],
            [pltpu.SemaphoreType.DMA for _ in range(2)],
            [pltpu.SemaphoreType.DMA for _ in range(2)],
            [pltpu.SemaphoreType.DMA for _ in range(2)],
            [pltpu.SemaphoreType.DMA for _ in range(2)],
        ],
    )
    return run(idx_flat, w_flat, embeddings)


# speedup vs baseline: 1.4942x; 1.0002x over previous
import jax
import jax.numpy as jnp
from jax import lax
from jax.experimental import pallas as pl
from jax.experimental.pallas import tpu as pltpu
from jax.experimental.pallas import tpu_sc as plsc

NUM_EMBEDDINGS = 1000000
D = 32
B = 16384
L = 50
NW = 32
BAGS_PER_W = B // NW
CHUNK_BAGS = 16
CHUNK_ROWS = CHUNK_BAGS * L
NCHUNK = BAGS_PER_W // CHUNK_BAGS
LANES = 16
NBUF = 4
DBLK = 8
UNROLL = 5


def _worker(idx_hbm, w_hbm, emb_hbm, out_hbm,
            idx_v, w_v, rows_v, out_v, isem, wsem, gsem, osem):
    cid = lax.axis_index("c")
    sid = lax.axis_index("s")
    wid = cid * 16 + sid
    lanes = lax.iota(jnp.int32, LANES)
    bag0 = wid * BAGS_PER_W

    def idx_copy(c, b):
        return pltpu.make_async_copy(
            idx_hbm.at[pl.ds((bag0 + c * CHUNK_BAGS) * L, CHUNK_ROWS)],
            idx_v[b], isem[b])

    def w_copy(c, b):
        return pltpu.make_async_copy(
            w_hbm.at[pl.ds((bag0 + c * CHUNK_BAGS) * L, CHUNK_ROWS)],
            w_v[b], wsem[b])

    def gather_copy(b):
        return pltpu.make_async_copy(emb_hbm.at[idx_v[b]], rows_v[b], gsem[b])

    def out_copy(c, b):
        return pltpu.make_async_copy(
            out_v[b],
            out_hbm.at[pl.ds(bag0 + c * CHUNK_BAGS, CHUNK_BAGS), :],
            osem[b])

    def compute(b):
        for g in range(CHUNK_BAGS // LANES):
            row_base = g * (LANES * L) + lanes * L
            bag_vec = g * LANES + lanes
            for d0 in range(0, D, DBLK):
                dsp = [(lanes + (d0 + k)) & (D - 1) for k in range(DBLK)]

                def body(i, accs):
                    new = list(accs)
                    for u in range(UNROLL):
                        r = row_base + (i * UNROLL + u)
                        wl = plsc.load_gather(w_v[b], [r])
                        vs = [plsc.load_gather(rows_v[b], [r, ds])
                              for ds in dsp]
                        for k in range(DBLK):
                            new[k] = new[k] + wl * vs[k]
                    return tuple(new)

                accs = lax.fori_loop(
                    0, L // UNROLL, body,
                    tuple(jnp.zeros((LANES,), jnp.float32)
                          for _ in range(DBLK)))

                for k in range(DBLK):
                    plsc.store_scatter(out_v[b], [bag_vec, dsp[k]], accs[k])

    for b in range(NBUF):
        idx_copy(b, b).start()
        idx_copy(b, b).wait()
        gather_copy(b).start()
        w_copy(b, b).start()

    def ring(cr, _):
        for sub in range(NBUF):
            c = cr * NBUF + sub
            b = sub
            gather_copy(b).wait()

            @pl.when(c + NBUF < NCHUNK)
            def _():
                idx_copy(c + NBUF, b).start()

            @pl.when(c >= NBUF)
            def _():
                out_copy(c, b).wait()

            w_copy(c, b).wait()
            compute(b)
            out_copy(c, b).start()

            @pl.when(c + NBUF < NCHUNK)
            def _():
                w_copy(c + NBUF, b).start()
                idx_copy(c + NBUF, b).wait()
                gather_copy(b).start()
        return ()

    lax.fori_loop(0, NCHUNK // NBUF, ring, ())
    for b in range(NBUF):
        out_copy(NCHUNK - NBUF + b, b).wait()


@jax.jit
def kernel(indices, weights, embeddings):
    idx_flat = indices.reshape(-1)
    w_flat = weights.reshape(-1)

    run = pl.kernel(
        _worker,
        out_type=jax.ShapeDtypeStruct((B, D), jnp.float32),
        mesh=plsc.VectorSubcoreMesh(core_axis_name="c", subcore_axis_name="s"),
        compiler_params=pltpu.CompilerParams(
            needs_layout_passes=False, use_tc_tiling_on_sc=False),
        scratch_types=[
            [pltpu.VMEM((CHUNK_ROWS,), jnp.int32) for _ in range(NBUF)],
            [pltpu.VMEM((CHUNK_ROWS,), jnp.float32) for _ in range(NBUF)],
            [pltpu.VMEM((CHUNK_ROWS, D), jnp.float32) for _ in range(NBUF)],
            [pltpu.VMEM((CHUNK_BAGS, D), jnp.float32) for _ in range(NBUF)],
            [pltpu.SemaphoreType.DMA for _ in range(NBUF)],
            [pltpu.SemaphoreType.DMA for _ in range(NBUF)],
            [pltpu.SemaphoreType.DMA for _ in range(NBUF)],
            [pltpu.SemaphoreType.DMA for _ in range(NBUF)],
        ],
    )
    return run(idx_flat, w_flat, embeddings)


# speedup vs baseline: 1.5281x; 1.0227x over previous
import jax
import jax.numpy as jnp
from jax import lax
from jax.experimental import pallas as pl
from jax.experimental.pallas import tpu as pltpu
from jax.experimental.pallas import tpu_sc as plsc

NUM_EMBEDDINGS = 1000000
D = 32
B = 16384
L = 50
NW = 32
BAGS_PER_W = B // NW
CHUNK_BAGS = 16
CHUNK_ROWS = CHUNK_BAGS * L
NCHUNK = BAGS_PER_W // CHUNK_BAGS
LANES = 16
NBUF = 4
DBLK = 8
UNROLL = 5


def _worker(idx_hbm, w_hbm, emb_hbm, out_hbm,
            idx_v, w_v, rows_v, out_v, isem, wsem, gsem, osem):
    cid = lax.axis_index("c")
    sid = lax.axis_index("s")
    wid = cid * 16 + sid
    lanes = lax.iota(jnp.int32, LANES)
    bag0 = wid * BAGS_PER_W

    def idx_copy(c, b):
        return pltpu.make_async_copy(
            idx_hbm.at[pl.ds((bag0 + c * CHUNK_BAGS) * L, CHUNK_ROWS)],
            idx_v[b], isem[b])

    def w_copy(c, b):
        return pltpu.make_async_copy(
            w_hbm.at[pl.ds((bag0 + c * CHUNK_BAGS) * L, CHUNK_ROWS)],
            w_v[b], wsem[b])

    def gather_copy(b):
        return pltpu.make_async_copy(emb_hbm.at[idx_v[b]], rows_v[b], gsem[b])

    def out_copy(c, b):
        return pltpu.make_async_copy(
            out_v[b],
            out_hbm.at[pl.ds(bag0 + c * CHUNK_BAGS, CHUNK_BAGS), :],
            osem[b])

    def compute(b):
        for g in range(CHUNK_BAGS // LANES):
            row_base = g * (LANES * L) + lanes * L
            bag_vec = g * LANES + lanes
            for d0 in range(0, D, DBLK):
                dsp = [(lanes + (d0 + k)) & (D - 1) for k in range(DBLK)]

                def body(i, accs):
                    new = list(accs)
                    for u in range(UNROLL):
                        r = row_base + (i * UNROLL + u)
                        wl = plsc.load_gather(w_v[b], [r])
                        vs = [plsc.load_gather(rows_v[b], [r, ds])
                              for ds in dsp]
                        for k in range(DBLK):
                            new[k] = new[k] + wl * vs[k]
                    return tuple(new)

                accs = lax.fori_loop(
                    0, L // UNROLL, body,
                    tuple(jnp.zeros((LANES,), jnp.float32)
                          for _ in range(DBLK)))

                for k in range(DBLK):
                    plsc.store_scatter(out_v[b], [bag_vec, dsp[k]], accs[k])

    for b in range(NBUF):
        idx_copy(b, b).start()
        idx_copy(b, b).wait()
        gather_copy(b).start()
        w_copy(b, b).start()

    def ring(cr, _):
        for sub in range(NBUF):
            c = cr * NBUF + sub
            b = sub
            gather_copy(b).wait()

            @pl.when(c + NBUF < NCHUNK)
            def _():
                idx_copy(c + NBUF, b).start()

            @pl.when(c >= NBUF)
            def _():
                out_copy(c, b).wait()

            w_copy(c, b).wait()
            compute(b)
            out_copy(c, b).start()

            @pl.when(c + NBUF < NCHUNK)
            def _():
                w_copy(c + NBUF, b).start()
                idx_copy(c + NBUF, b).wait()
                gather_copy(b).start()
        return ()

    lax.fori_loop(0, NCHUNK // NBUF, ring, ())
    for b in range(NBUF):
        out_copy(NCHUNK - NBUF + b, b).wait()


@jax.jit
def kernel(indices, weights, embeddings):
    idx_flat = indices.reshape(-1) * 4
    w_flat = weights.reshape(-1)
    emb_pad = jnp.pad(embeddings, ((0, 0), (0, 128 - D)))
    emb_rows = emb_pad.reshape(4 * NUM_EMBEDDINGS, D)

    run = pl.kernel(
        _worker,
        out_type=jax.ShapeDtypeStruct((B, D), jnp.float32),
        mesh=plsc.VectorSubcoreMesh(core_axis_name="c", subcore_axis_name="s"),
        compiler_params=pltpu.CompilerParams(
            needs_layout_passes=False, use_tc_tiling_on_sc=False),
        scratch_types=[
            [pltpu.VMEM((CHUNK_ROWS,), jnp.int32) for _ in range(NBUF)],
            [pltpu.VMEM((CHUNK_ROWS,), jnp.float32) for _ in range(NBUF)],
            [pltpu.VMEM((CHUNK_ROWS, D), jnp.float32) for _ in range(NBUF)],
            [pltpu.VMEM((CHUNK_BAGS, D), jnp.float32) for _ in range(NBUF)],
            [pltpu.SemaphoreType.DMA for _ in range(NBUF)],
            [pltpu.SemaphoreType.DMA for _ in range(NBUF)],
            [pltpu.SemaphoreType.DMA for _ in range(NBUF)],
            [pltpu.SemaphoreType.DMA for _ in range(NBUF)],
        ],
    )
    return run(idx_flat, w_flat, emb_rows)
